# Initial kernel scaffold; baseline (speedup 1.0000x reference)
import jax
import jax.numpy as jnp
from jax.experimental import pallas as pl


def kernel(params, tokens, cu_seqlens):
    raise NotImplementedError("write your pallas kernel here")


# speedup vs baseline: 2.5346x; 2.5346x over previous
import functools

import jax
import jax.numpy as jnp
from jax.experimental import pallas as pl

B = 4
LMAX = 512
D = 1024
H = 16
HD = 64
V = 30
FF = 4096
FF_BLK = 1024


def _ln(x, g, b):
    mu = jnp.mean(x, axis=-1, keepdims=True)
    var = jnp.mean((x - mu) ** 2, axis=-1, keepdims=True)
    return (x - mu) * jax.lax.rsqrt(var + 1e-5) * g + b


def _embed_kernel(tok_ref, emb_ref, o_ref):
    cls = jax.lax.broadcasted_iota(jnp.int32, (tok_ref.shape[0], V), 1)
    onehot = (tok_ref[:] == cls).astype(jnp.float32)
    o_ref[:] = jnp.dot(onehot, emb_ref[:], preferred_element_type=jnp.float32)


def _attn_kernel(x_ref, segr_ref, segc_ref, cos_ref, sin_ref, wq_ref, wk_ref,
                 wv_ref, wo_ref, g_ref, b_ref, o_ref):
    x = x_ref[:]
    n = x.shape[0]
    h = _ln(x, g_ref[:], b_ref[:])
    q = jnp.dot(h, wq_ref[:], preferred_element_type=jnp.float32)
    k = jnp.dot(h, wk_ref[:], preferred_element_type=jnp.float32)
    v = jnp.dot(h, wv_ref[:], preferred_element_type=jnp.float32)
    cos = cos_ref[:]
    sin = sin_ref[:]
    rowi = jax.lax.broadcasted_iota(jnp.int32, (n, n), 0)
    coli = jax.lax.broadcasted_iota(jnp.int32, (n, n), 1)
    mask = (rowi >= coli) & (segr_ref[:] == segc_ref[:])
    scale = 1.0 / (HD ** 0.5)
    o_cols = []
    for hh in range(H):
        sl = slice(hh * HD, (hh + 1) * HD)
        qh = q[:, sl]
        kh = k[:, sl]
        q1, q2 = qh[:, :HD // 2], qh[:, HD // 2:]
        k1, k2 = kh[:, :HD // 2], kh[:, HD // 2:]
        qr = jnp.concatenate([q1 * cos - q2 * sin, q1 * sin + q2 * cos], axis=1)
        kr = jnp.concatenate([k1 * cos - k2 * sin, k1 * sin + k2 * cos], axis=1)
        s = jax.lax.dot_general(qr, kr, (((1,), (1,)), ((), ())),
                                preferred_element_type=jnp.float32) * scale
        s = jnp.where(mask, s, -1e9)
        m = jnp.max(s, axis=1, keepdims=True)
        p = jnp.exp(s - m)
        a = p / jnp.sum(p, axis=1, keepdims=True)
        o_cols.append(jnp.dot(a, v[:, sl], preferred_element_type=jnp.float32))
    o = jnp.concatenate(o_cols, axis=1)
    o_ref[:] = x + jnp.dot(o, wo_ref[:], preferred_element_type=jnp.float32)


def _ffn_kernel(x_ref, g_ref, b_ref, w1_ref, w2_ref, o_ref):
    step = pl.program_id(0)
    h = _ln(x_ref[:], g_ref[:], b_ref[:])
    mid = jax.nn.gelu(jnp.dot(h, w1_ref[:], preferred_element_type=jnp.float32))
    contrib = jnp.dot(mid, w2_ref[:], preferred_element_type=jnp.float32)

    @pl.when(step == 0)
    def _():
        o_ref[:] = x_ref[:] + contrib

    @pl.when(step != 0)
    def _():
        o_ref[:] = o_ref[:] + contrib


def _final_kernel(x_ref, g_ref, b_ref, w_ref, o_ref):
    h = _ln(x_ref[:], g_ref[:], b_ref[:])
    o_ref[:] = jnp.dot(h, w_ref[:], preferred_element_type=jnp.float32)


def kernel(params, tokens, cu_seqlens):
    T = tokens.shape[0]
    f32 = jnp.float32

    idx = jnp.arange(T, dtype=jnp.int32)
    seg = jnp.searchsorted(cu_seqlens, idx, side='right').astype(jnp.int32) - 1
    offs = idx - cu_seqlens[seg]

    half = HD // 2
    inv = 1.0 / (10000.0 ** (jnp.arange(half, dtype=f32) / half))
    ang = offs.astype(f32)[:, None] * inv[None, :]
    cos = jnp.cos(ang)
    sin = jnp.sin(ang)

    perm = (jnp.arange(D).reshape(H, half, 2).transpose(0, 2, 1).reshape(D))

    x = pl.pallas_call(
        _embed_kernel,
        out_shape=jax.ShapeDtypeStruct((T, D), f32),
    )(tokens.reshape(T, 1), params['embed'])

    for lp in params['layers']:
        wq_p = lp['wq'][:, perm]
        wk_p = lp['wk'][:, perm]
        x = pl.pallas_call(
            _attn_kernel,
            out_shape=jax.ShapeDtypeStruct((T, D), f32),
        )(x, seg.reshape(T, 1), seg.reshape(1, T), cos, sin, wq_p, wk_p,
          lp['wv'], lp['wo'],
          lp['n1g'].reshape(1, D), lp['n1b'].reshape(1, D))

        nblk = FF // FF_BLK
        x = pl.pallas_call(
            _ffn_kernel,
            grid=(nblk,),
            in_specs=[
                pl.BlockSpec((T, D), lambda i: (0, 0)),
                pl.BlockSpec((1, D), lambda i: (0, 0)),
                pl.BlockSpec((1, D), lambda i: (0, 0)),
                pl.BlockSpec((D, FF_BLK), lambda i: (0, i)),
                pl.BlockSpec((FF_BLK, D), lambda i: (i, 0)),
            ],
            out_specs=pl.BlockSpec((T, D), lambda i: (0, 0)),
            out_shape=jax.ShapeDtypeStruct((T, D), f32),
        )(x, lp['n2g'].reshape(1, D), lp['n2b'].reshape(1, D),
          lp['w1'], lp['w2'])

    logits = pl.pallas_call(
        _final_kernel,
        out_shape=jax.ShapeDtypeStruct((T, V), f32),
    )(x, params['nfg'].reshape(1, D), params['nfb'].reshape(1, D),
      params['out_w'])
    return logits
